# Initial kernel scaffold; baseline (speedup 1.0000x reference)
import jax
import jax.numpy as jnp
from jax.experimental import pallas as pl


def kernel(x_seq):
    raise NotImplementedError("write your pallas kernel here")


# speedup vs baseline: 1.5632x; 1.5632x over previous
import jax
import jax.numpy as jnp
import numpy as np
from jax.experimental import pallas as pl
from jax.experimental.pallas import tpu as pltpu

_N = 16384
_D = 2048
_R = 256

_RATE_THRESH = 2516582


def _threefry_bits(k0, k1, cnt):
    ks0 = k0
    ks1 = k1
    ks2 = k0 ^ k1 ^ jnp.uint32(0x1BD11BDA)
    ks = (ks0, ks1, ks2)
    rotations = ((13, 15, 26, 6), (17, 29, 16, 24))

    def rotl(v, r):
        return (v << jnp.uint32(r)) | (v >> jnp.uint32(32 - r))

    x0 = jnp.broadcast_to(ks0, cnt.shape)
    x1 = cnt + ks1
    for i in range(5):
        for r in rotations[i % 2]:
            x0 = x0 + x1
            x1 = rotl(x1, r)
            x1 = x0 ^ x1
        x0 = x0 + ks[(i + 1) % 3]
        x1 = x1 + ks[(i + 2) % 3] + jnp.uint32(i + 1)
    return x0 ^ x1


_SUB = 8


def _mask_body(key_ref, x_ref, flags_ref, out_x_ref, out_m_ref):
    i = pl.program_id(0)
    k0 = key_ref[0].astype(jnp.uint32)
    k1 = key_ref[1].astype(jnp.uint32)
    rows = jax.lax.broadcasted_iota(jnp.uint32, (_SUB, _D), 0)
    cols = jax.lax.broadcasted_iota(jnp.uint32, (_SUB, _D), 1)
    tile = rows * jnp.uint32(_D) + cols

    def step(t, carry):
        r0 = t * _SUB
        base = (i * (_R * _D)).astype(jnp.uint32) + (r0 * _D).astype(jnp.uint32)
        cnt = base + tile
        bits = _threefry_bits(k0, k1, cnt)
        feat = (bits >> jnp.uint32(9)).astype(jnp.int32) <= _RATE_THRESH
        rowflag = flags_ref[pl.ds(r0, _SUB), :] != 0
        mask = feat & rowflag
        out_m_ref[pl.ds(r0, _SUB), :] = mask.astype(jnp.int32)
        out_x_ref[pl.ds(r0, _SUB), :] = jnp.where(
            mask, jnp.float32(0.0), x_ref[pl.ds(r0, _SUB), :]
        )
        return carry

    jax.lax.fori_loop(0, _R // _SUB, step, 0)


def kernel(x_seq):
    n, d = x_seq.shape
    kperm, kmask = jax.random.split(jax.random.key(1))
    key_data = jax.random.key_data(kmask).astype(jnp.int32)
    perm = jax.random.permutation(kperm, n)
    masked_rows = perm[: n // 2]
    flags = jnp.zeros((n, 1), jnp.int32).at[masked_rows].set(1)

    masked_x, input_mask = pl.pallas_call(
        _mask_body,
        grid=(n // _R,),
        in_specs=[
            pl.BlockSpec(memory_space=pltpu.SMEM),
            pl.BlockSpec((_R, _D), lambda i: (i, 0)),
            pl.BlockSpec((_R, 1), lambda i: (i, 0)),
        ],
        out_specs=[
            pl.BlockSpec((_R, _D), lambda i: (i, 0)),
            pl.BlockSpec((_R, _D), lambda i: (i, 0)),
        ],
        out_shape=[
            jax.ShapeDtypeStruct((n, d), jnp.float32),
            jax.ShapeDtypeStruct((n, d), jnp.int32),
        ],
        compiler_params=pltpu.CompilerParams(
            dimension_semantics=("arbitrary",),
        ),
    )(key_data, x_seq, flags)
    return masked_x, input_mask
